# baseline (speedup 1.0000x reference)
import functools

import jax
import jax.numpy as jnp
from jax import lax
from jax.experimental import pallas as pl
from jax.experimental.pallas import tpu as pltpu
from jax.experimental.pallas import tpu_sc as plsc

N = 10000
NP = 10240
E = 320000
G = 64
DIN = 128
H = 256
NCLS = 10

CH = 128
EP = 327680
NB = EP // CH
CORES = 2
SUBC = 16
BN = 1000

_MESH = plsc.VectorSubcoreMesh(
    core_axis_name="c", subcore_axis_name="s", num_cores=CORES, num_subcores=SUBC
)


def _make_sc_agg(table_len, chunks_per_tile, core_offset):
    npt = chunks_per_tile
    assert npt % 4 == 0

    @functools.partial(
        pl.kernel,
        out_type=jax.ShapeDtypeStruct((CORES, NP, DIN), jnp.float32),
        mesh=_MESH,
        scratch_types=[
            pltpu.VMEM_SHARED((NP, DIN), jnp.float32),
            pltpu.VMEM((CH,), jnp.int32),
            pltpu.VMEM((CH,), jnp.int32),
            pltpu.VMEM((CH, DIN), jnp.float32),
            pltpu.SemaphoreType.DMA,
        ],
    )
    def k(tab_hbm, e_hbm, z_hbm, out_hbm, acc, sbuf, dbuf, rows, sem):
        c = lax.axis_index("c")
        s = lax.axis_index("s")
        rpt = NP // SUBC
        pltpu.sync_copy(z_hbm.at[pl.ds(s * rpt, rpt)], acc.at[pl.ds(s * rpt, rpt)])
        if core_offset:
            base = s * npt
            soff = c * NB
        else:
            base = (c * SUBC + s) * npt
            soff = 0
        plsc.subcore_barrier()

        def step(j, carry):
            row = soff + base + j
            pltpu.sync_copy(e_hbm.at[row, 0], sbuf)
            pltpu.sync_copy(e_hbm.at[row, 1], dbuf)
            pltpu.async_copy(tab_hbm.at[sbuf], rows, sem).wait()
            pltpu.sync_copy(rows, acc.at[dbuf], add=True)
            return carry

        lax.fori_loop(0, npt, step, 0)
        plsc.subcore_barrier()
        pltpu.sync_copy(acc.at[pl.ds(s * rpt, rpt)], out_hbm.at[c].at[pl.ds(s * rpt, rpt)])

    return k


_sc_agg1 = _make_sc_agg(NP, NB // (CORES * SUBC), core_offset=False)
_sc_agg2 = _make_sc_agg(2 * NP, NB // SUBC, core_offset=True)


def _mlp1_body(x_ref, p_ref, w1_ref, b1_ref, w2_ref, b2_ref, o_ref):
    h = x_ref[...] + p_ref[0] + p_ref[1]
    t = jnp.dot(h, w1_ref[...], preferred_element_type=jnp.float32) + b1_ref[...]
    t = jnp.maximum(t, 0.0)
    u = jnp.dot(t, w2_ref[...], preferred_element_type=jnp.float32) + b2_ref[...]
    u = jnp.where(u >= 0, u, 0.01 * u)
    o_ref[0] = u[:, :DIN]
    o_ref[1] = u[:, DIN:]


def _tc_mlp1(x, p, w1, b1, w2, b2):
    return pl.pallas_call(
        _mlp1_body,
        grid=(N // BN,),
        in_specs=[
            pl.BlockSpec((BN, DIN), lambda i: (i, 0)),
            pl.BlockSpec((CORES, BN, DIN), lambda i: (0, i, 0)),
            pl.BlockSpec((DIN, H), lambda i: (0, 0)),
            pl.BlockSpec((1, H), lambda i: (0, 0)),
            pl.BlockSpec((H, H), lambda i: (0, 0)),
            pl.BlockSpec((1, H), lambda i: (0, 0)),
        ],
        out_specs=pl.BlockSpec((CORES, BN, DIN), lambda i: (0, i, 0)),
        out_shape=jax.ShapeDtypeStruct((CORES, NP, DIN), jnp.float32),
    )(x, p, w1, b1, w2, b2)


def _mlp2_body(h_ref, a_ref, b_ref, w1_ref, b1_ref, w2_ref, b2_ref,
               lw_ref, lb_ref, l2w_ref, l2b_ref, l3w_ref, l3b_ref,
               o_ref, sums, cnts):
    i = pl.program_id(0)
    z = jnp.concatenate([h_ref[0] + a_ref[0], h_ref[1] + a_ref[1]], axis=1)
    t = jnp.dot(z, w1_ref[...], preferred_element_type=jnp.float32) + b1_ref[...]
    t = jnp.maximum(t, 0.0)
    u = jnp.dot(t, w2_ref[...], preferred_element_type=jnp.float32) + b2_ref[...]
    u = jnp.where(u >= 0, u, 0.01 * u)
    bvec = b_ref[0, 0]
    oh = (bvec[:, None] == lax.broadcasted_iota(jnp.int32, (BN, G), 1)).astype(jnp.float32)
    ps = lax.dot_general(oh, u, (((0,), (0,)), ((), ())),
                         preferred_element_type=jnp.float32)
    pc = lax.dot_general(oh, jnp.ones((BN, 1), jnp.float32), (((0,), (0,)), ((), ())),
                         preferred_element_type=jnp.float32)

    @pl.when(i == 0)
    def _():
        sums[...] = ps
        cnts[...] = pc

    @pl.when(i != 0)
    def _():
        sums[...] += ps
        cnts[...] += pc

    @pl.when(i == pl.num_programs(0) - 1)
    def _():
        cnt = jnp.maximum(cnts[...], 1.0)
        mean = sums[...] / cnt
        q = jnp.dot(mean, lw_ref[...], preferred_element_type=jnp.float32) + lb_ref[...]
        q = jnp.where(q >= 0, q, 0.01 * q)
        q = jnp.dot(q, l2w_ref[...], preferred_element_type=jnp.float32) + l2b_ref[...]
        q = jnp.where(q >= 0, q, 0.01 * q)
        o_ref[...] = jnp.dot(q, l3w_ref[...], preferred_element_type=jnp.float32) + l3b_ref[...]


def _tc_mlp2(h, a, batch_r, w1, b1, w2, b2, lw, lb, l2w, l2b, l3w, l3b):
    full = lambda shp: pl.BlockSpec(shp, lambda i: tuple(0 for _ in shp))
    return pl.pallas_call(
        _mlp2_body,
        grid=(N // BN,),
        in_specs=[
            pl.BlockSpec((CORES, BN, DIN), lambda i: (0, i, 0)),
            pl.BlockSpec((CORES, BN, DIN), lambda i: (0, i, 0)),
            pl.BlockSpec((1, 1, BN), lambda i: (i, 0, 0)),
            full((H, H)), full((1, H)), full((H, H)), full((1, H)),
            full((H, H // 2)), full((1, H // 2)),
            full((H // 2, H // 2)), full((1, H // 2)),
            full((H // 2, NCLS)), full((1, NCLS)),
        ],
        out_specs=pl.BlockSpec((G, NCLS), lambda i: (0, 0)),
        out_shape=jax.ShapeDtypeStruct((G, NCLS), jnp.float32),
        scratch_shapes=[
            pltpu.VMEM((G, H), jnp.float32),
            pltpu.VMEM((G, 1), jnp.float32),
        ],
    )(h, a, batch_r, w1, b1, w2, b2, lw, lb, l2w, l2b, l3w, l3b)


def kernel(x, edge_index, batch, c1_w1, c1_b1, c1_w2, c1_b2,
           c2_w1, c2_b1, c2_w2, c2_b2, lin_w, lin_b, lin2_w, lin2_b,
           lin3_w, lin3_b):
    pad = EP - E
    src = jnp.concatenate([edge_index[0], jnp.zeros((pad,), jnp.int32)]).reshape(NB, CH)
    dst = jnp.concatenate([edge_index[1], jnp.full((pad,), N, jnp.int32)]).reshape(NB, CH)
    eflat = jnp.concatenate(
        [jnp.stack([src, dst], 1), jnp.stack([src + NP, dst], 1)], axis=0)
    zeros = jnp.zeros((NP, DIN), jnp.float32)

    p = _sc_agg1(x, eflat, zeros)
    h1 = _tc_mlp1(x, p, c1_w1, c1_b1.reshape(1, H), c1_w2, c1_b2.reshape(1, H))
    a2 = _sc_agg2(h1.reshape(2 * NP, DIN), eflat, zeros)
    batch_r = batch.reshape(N // BN, 1, BN)
    return _tc_mlp2(
        h1, a2, batch_r,
        c2_w1, c2_b1.reshape(1, H), c2_w2, c2_b2.reshape(1, H),
        lin_w, lin_b.reshape(1, H // 2),
        lin2_w, lin2_b.reshape(1, H // 2),
        lin3_w, lin3_b.reshape(1, NCLS),
    )


# speedup vs baseline: 1.4980x; 1.4980x over previous
import functools

import jax
import jax.numpy as jnp
from jax import lax
from jax.experimental import pallas as pl
from jax.experimental.pallas import tpu as pltpu
from jax.experimental.pallas import tpu_sc as plsc

N = 10000
NP = 10240
E = 320000
G = 64
DIN = 128
H = 256
NCLS = 10

CH = 120
EP = 322560
NB = EP // CH
CORES = 2
SUBC = 16
BN = 1000

_MESH = plsc.VectorSubcoreMesh(
    core_axis_name="c", subcore_axis_name="s", num_cores=CORES, num_subcores=SUBC
)


def _make_sc_agg(table_len, chunks_per_tile, core_offset):
    npt = chunks_per_tile
    assert npt % 4 == 0

    @functools.partial(
        pl.kernel,
        out_type=jax.ShapeDtypeStruct((CORES, NP, DIN), jnp.float32),
        mesh=_MESH,
        scratch_types=[
            pltpu.VMEM_SHARED((NP, DIN), jnp.float32),
            pltpu.VMEM((CH,), jnp.int32),
            pltpu.VMEM((CH,), jnp.int32),
            pltpu.VMEM((CH, DIN), jnp.float32),
            pltpu.SemaphoreType.DMA,
        ],
    )
    def k(tab_hbm, e_hbm, z_hbm, out_hbm, acc, sbuf, dbuf, rows, sem):
        c = lax.axis_index("c")
        s = lax.axis_index("s")
        rpt = NP // SUBC
        pltpu.sync_copy(z_hbm.at[pl.ds(s * rpt, rpt)], acc.at[pl.ds(s * rpt, rpt)])
        if core_offset:
            base = s * npt
            soff = c * NB
        else:
            base = (c * SUBC + s) * npt
            soff = 0
        plsc.subcore_barrier()

        def step(j, carry):
            row = soff + base + j
            pltpu.sync_copy(e_hbm.at[row, 0], sbuf)
            pltpu.sync_copy(e_hbm.at[row, 1], dbuf)
            pltpu.async_copy(tab_hbm.at[sbuf], rows, sem).wait()
            pltpu.sync_copy(rows, acc.at[dbuf], add=True)
            return carry

        lax.fori_loop(0, npt, step, 0)
        plsc.subcore_barrier()
        pltpu.sync_copy(acc.at[pl.ds(s * rpt, rpt)], out_hbm.at[c].at[pl.ds(s * rpt, rpt)])

    return k


_sc_agg1 = _make_sc_agg(NP, NB // (CORES * SUBC), core_offset=False)
_sc_agg2 = _make_sc_agg(2 * NP, NB // SUBC, core_offset=True)


def _mlp1_body(x_ref, p_ref, w1_ref, b1_ref, w2_ref, b2_ref, o_ref):
    h = x_ref[...] + p_ref[0] + p_ref[1]
    t = jnp.dot(h, w1_ref[...], preferred_element_type=jnp.float32) + b1_ref[...]
    t = jnp.maximum(t, 0.0)
    u = jnp.dot(t, w2_ref[...], preferred_element_type=jnp.float32) + b2_ref[...]
    u = jnp.where(u >= 0, u, 0.01 * u)
    o_ref[0] = u[:, :DIN]
    o_ref[1] = u[:, DIN:]


def _tc_mlp1(x, p, w1, b1, w2, b2):
    return pl.pallas_call(
        _mlp1_body,
        grid=(N // BN,),
        in_specs=[
            pl.BlockSpec((BN, DIN), lambda i: (i, 0)),
            pl.BlockSpec((CORES, BN, DIN), lambda i: (0, i, 0)),
            pl.BlockSpec((DIN, H), lambda i: (0, 0)),
            pl.BlockSpec((1, H), lambda i: (0, 0)),
            pl.BlockSpec((H, H), lambda i: (0, 0)),
            pl.BlockSpec((1, H), lambda i: (0, 0)),
        ],
        out_specs=pl.BlockSpec((CORES, BN, DIN), lambda i: (0, i, 0)),
        out_shape=jax.ShapeDtypeStruct((CORES, NP, DIN), jnp.float32),
    )(x, p, w1, b1, w2, b2)


def _mlp2_body(h_ref, a_ref, b_ref, w1_ref, b1_ref, w2_ref, b2_ref,
               lw_ref, lb_ref, l2w_ref, l2b_ref, l3w_ref, l3b_ref,
               o_ref, sums, cnts):
    i = pl.program_id(0)
    z = jnp.concatenate([h_ref[0] + a_ref[0], h_ref[1] + a_ref[1]], axis=1)
    t = jnp.dot(z, w1_ref[...], preferred_element_type=jnp.float32) + b1_ref[...]
    t = jnp.maximum(t, 0.0)
    u = jnp.dot(t, w2_ref[...], preferred_element_type=jnp.float32) + b2_ref[...]
    u = jnp.where(u >= 0, u, 0.01 * u)
    bvec = b_ref[0, 0]
    oh = (bvec[:, None] == lax.broadcasted_iota(jnp.int32, (BN, G), 1)).astype(jnp.float32)
    ps = lax.dot_general(oh, u, (((0,), (0,)), ((), ())),
                         preferred_element_type=jnp.float32)
    pc = lax.dot_general(oh, jnp.ones((BN, 1), jnp.float32), (((0,), (0,)), ((), ())),
                         preferred_element_type=jnp.float32)

    @pl.when(i == 0)
    def _():
        sums[...] = ps
        cnts[...] = pc

    @pl.when(i != 0)
    def _():
        sums[...] += ps
        cnts[...] += pc

    @pl.when(i == pl.num_programs(0) - 1)
    def _():
        cnt = jnp.maximum(cnts[...], 1.0)
        mean = sums[...] / cnt
        q = jnp.dot(mean, lw_ref[...], preferred_element_type=jnp.float32) + lb_ref[...]
        q = jnp.where(q >= 0, q, 0.01 * q)
        q = jnp.dot(q, l2w_ref[...], preferred_element_type=jnp.float32) + l2b_ref[...]
        q = jnp.where(q >= 0, q, 0.01 * q)
        o_ref[...] = jnp.dot(q, l3w_ref[...], preferred_element_type=jnp.float32) + l3b_ref[...]


def _tc_mlp2(h, a, batch_r, w1, b1, w2, b2, lw, lb, l2w, l2b, l3w, l3b):
    full = lambda shp: pl.BlockSpec(shp, lambda i: tuple(0 for _ in shp))
    return pl.pallas_call(
        _mlp2_body,
        grid=(N // BN,),
        in_specs=[
            pl.BlockSpec((CORES, BN, DIN), lambda i: (0, i, 0)),
            pl.BlockSpec((CORES, BN, DIN), lambda i: (0, i, 0)),
            pl.BlockSpec((1, 1, BN), lambda i: (i, 0, 0)),
            full((H, H)), full((1, H)), full((H, H)), full((1, H)),
            full((H, H // 2)), full((1, H // 2)),
            full((H // 2, H // 2)), full((1, H // 2)),
            full((H // 2, NCLS)), full((1, NCLS)),
        ],
        out_specs=pl.BlockSpec((G, NCLS), lambda i: (0, 0)),
        out_shape=jax.ShapeDtypeStruct((G, NCLS), jnp.float32),
        scratch_shapes=[
            pltpu.VMEM((G, H), jnp.float32),
            pltpu.VMEM((G, 1), jnp.float32),
        ],
    )(h, a, batch_r, w1, b1, w2, b2, lw, lb, l2w, l2b, l3w, l3b)


def kernel(x, edge_index, batch, c1_w1, c1_b1, c1_w2, c1_b2,
           c2_w1, c2_b1, c2_w2, c2_b2, lin_w, lin_b, lin2_w, lin2_b,
           lin3_w, lin3_b):
    pad = EP - E
    src = jnp.concatenate([edge_index[0], jnp.zeros((pad,), jnp.int32)]).reshape(NB, CH)
    dst = jnp.concatenate([edge_index[1], jnp.full((pad,), N, jnp.int32)]).reshape(NB, CH)
    eflat = jnp.concatenate(
        [jnp.stack([src, dst], 1), jnp.stack([src + NP, dst], 1)], axis=0)
    zeros = jnp.zeros((NP, DIN), jnp.float32)

    p = _sc_agg1(x, eflat, zeros)
    h1 = _tc_mlp1(x, p, c1_w1, c1_b1.reshape(1, H), c1_w2, c1_b2.reshape(1, H))
    a2 = _sc_agg2(h1.reshape(2 * NP, DIN), eflat, zeros)
    batch_r = batch.reshape(N // BN, 1, BN)
    return _tc_mlp2(
        h1, a2, batch_r,
        c2_w1, c2_b1.reshape(1, H), c2_w2, c2_b2.reshape(1, H),
        lin_w, lin_b.reshape(1, H // 2),
        lin2_w, lin2_b.reshape(1, H // 2),
        lin3_w, lin3_b.reshape(1, NCLS),
    )


# speedup vs baseline: 1.5211x; 1.0154x over previous
import functools

import jax
import jax.numpy as jnp
from jax import lax
from jax.experimental import pallas as pl
from jax.experimental.pallas import tpu as pltpu
from jax.experimental.pallas import tpu_sc as plsc

N = 10000
NP = 10240
E = 320000
G = 64
DIN = 128
H = 256
NCLS = 10

CH = 80
EP = 320000
NB = EP // CH
CORES = 2
SUBC = 16
BN = 1000

_MESH = plsc.VectorSubcoreMesh(
    core_axis_name="c", subcore_axis_name="s", num_cores=CORES, num_subcores=SUBC
)


def _make_sc_agg(table_len, chunks_per_tile, core_offset):
    npt = chunks_per_tile

    @functools.partial(
        pl.kernel,
        out_type=jax.ShapeDtypeStruct((CORES, NP, DIN), jnp.float32),
        mesh=_MESH,
        scratch_types=[
            pltpu.VMEM_SHARED((NP, DIN), jnp.float32),
            pltpu.VMEM((CH,), jnp.int32),
            pltpu.VMEM((CH,), jnp.int32),
            pltpu.VMEM((CH, DIN), jnp.float32),
            pltpu.SemaphoreType.DMA,
        ],
    )
    def k(tab_hbm, e_hbm, z_hbm, out_hbm, acc, sbuf, dbuf, rows, sem):
        c = lax.axis_index("c")
        s = lax.axis_index("s")
        rpt = NP // SUBC
        pltpu.sync_copy(z_hbm.at[pl.ds(s * rpt, rpt)], acc.at[pl.ds(s * rpt, rpt)])
        if core_offset:
            base = s * npt
            soff = c * NB
        else:
            base = (c * SUBC + s) * npt
            soff = 0
        plsc.subcore_barrier()

        def step(j, carry):
            row = soff + base + j
            pltpu.sync_copy(e_hbm.at[row, 0], sbuf)
            pltpu.sync_copy(e_hbm.at[row, 1], dbuf)
            pltpu.async_copy(tab_hbm.at[sbuf], rows, sem).wait()
            pltpu.sync_copy(rows, acc.at[dbuf], add=True)
            return carry

        lax.fori_loop(0, npt, step, 0)
        plsc.subcore_barrier()
        pltpu.sync_copy(acc.at[pl.ds(s * rpt, rpt)], out_hbm.at[c].at[pl.ds(s * rpt, rpt)])

    return k


_sc_agg1 = _make_sc_agg(NP, NB // (CORES * SUBC), core_offset=False)
_sc_agg2 = _make_sc_agg(2 * NP, NB // SUBC, core_offset=True)


def _mlp1_body(x_ref, p_ref, w1_ref, b1_ref, w2_ref, b2_ref, o_ref):
    h = x_ref[...] + p_ref[0] + p_ref[1]
    t = jnp.dot(h, w1_ref[...], preferred_element_type=jnp.float32) + b1_ref[...]
    t = jnp.maximum(t, 0.0)
    u = jnp.dot(t, w2_ref[...], preferred_element_type=jnp.float32) + b2_ref[...]
    u = jnp.where(u >= 0, u, 0.01 * u)
    o_ref[0] = u[:, :DIN]
    o_ref[1] = u[:, DIN:]


def _tc_mlp1(x, p, w1, b1, w2, b2):
    return pl.pallas_call(
        _mlp1_body,
        grid=(N // BN,),
        in_specs=[
            pl.BlockSpec((BN, DIN), lambda i: (i, 0)),
            pl.BlockSpec((CORES, BN, DIN), lambda i: (0, i, 0)),
            pl.BlockSpec((DIN, H), lambda i: (0, 0)),
            pl.BlockSpec((1, H), lambda i: (0, 0)),
            pl.BlockSpec((H, H), lambda i: (0, 0)),
            pl.BlockSpec((1, H), lambda i: (0, 0)),
        ],
        out_specs=pl.BlockSpec((CORES, BN, DIN), lambda i: (0, i, 0)),
        out_shape=jax.ShapeDtypeStruct((CORES, NP, DIN), jnp.float32),
    )(x, p, w1, b1, w2, b2)


def _mlp2_body(h_ref, a_ref, b_ref, w1_ref, b1_ref, w2_ref, b2_ref,
               lw_ref, lb_ref, l2w_ref, l2b_ref, l3w_ref, l3b_ref,
               o_ref, sums, cnts):
    i = pl.program_id(0)
    z = jnp.concatenate([h_ref[0] + a_ref[0], h_ref[1] + a_ref[1]], axis=1)
    t = jnp.dot(z, w1_ref[...], preferred_element_type=jnp.float32) + b1_ref[...]
    t = jnp.maximum(t, 0.0)
    u = jnp.dot(t, w2_ref[...], preferred_element_type=jnp.float32) + b2_ref[...]
    u = jnp.where(u >= 0, u, 0.01 * u)
    bvec = b_ref[0, 0]
    oh = (bvec[:, None] == lax.broadcasted_iota(jnp.int32, (BN, G), 1)).astype(jnp.float32)
    ps = lax.dot_general(oh, u, (((0,), (0,)), ((), ())),
                         preferred_element_type=jnp.float32)
    pc = lax.dot_general(oh, jnp.ones((BN, 1), jnp.float32), (((0,), (0,)), ((), ())),
                         preferred_element_type=jnp.float32)

    @pl.when(i == 0)
    def _():
        sums[...] = ps
        cnts[...] = pc

    @pl.when(i != 0)
    def _():
        sums[...] += ps
        cnts[...] += pc

    @pl.when(i == pl.num_programs(0) - 1)
    def _():
        cnt = jnp.maximum(cnts[...], 1.0)
        mean = sums[...] / cnt
        q = jnp.dot(mean, lw_ref[...], preferred_element_type=jnp.float32) + lb_ref[...]
        q = jnp.where(q >= 0, q, 0.01 * q)
        q = jnp.dot(q, l2w_ref[...], preferred_element_type=jnp.float32) + l2b_ref[...]
        q = jnp.where(q >= 0, q, 0.01 * q)
        o_ref[...] = jnp.dot(q, l3w_ref[...], preferred_element_type=jnp.float32) + l3b_ref[...]


def _tc_mlp2(h, a, batch_r, w1, b1, w2, b2, lw, lb, l2w, l2b, l3w, l3b):
    full = lambda shp: pl.BlockSpec(shp, lambda i: tuple(0 for _ in shp))
    return pl.pallas_call(
        _mlp2_body,
        grid=(N // BN,),
        in_specs=[
            pl.BlockSpec((CORES, BN, DIN), lambda i: (0, i, 0)),
            pl.BlockSpec((CORES, BN, DIN), lambda i: (0, i, 0)),
            pl.BlockSpec((1, 1, BN), lambda i: (i, 0, 0)),
            full((H, H)), full((1, H)), full((H, H)), full((1, H)),
            full((H, H // 2)), full((1, H // 2)),
            full((H // 2, H // 2)), full((1, H // 2)),
            full((H // 2, NCLS)), full((1, NCLS)),
        ],
        out_specs=pl.BlockSpec((G, NCLS), lambda i: (0, 0)),
        out_shape=jax.ShapeDtypeStruct((G, NCLS), jnp.float32),
        scratch_shapes=[
            pltpu.VMEM((G, H), jnp.float32),
            pltpu.VMEM((G, 1), jnp.float32),
        ],
    )(h, a, batch_r, w1, b1, w2, b2, lw, lb, l2w, l2b, l3w, l3b)


def kernel(x, edge_index, batch, c1_w1, c1_b1, c1_w2, c1_b2,
           c2_w1, c2_b1, c2_w2, c2_b2, lin_w, lin_b, lin2_w, lin2_b,
           lin3_w, lin3_b):
    pad = EP - E
    src = jnp.concatenate([edge_index[0], jnp.zeros((pad,), jnp.int32)]).reshape(NB, CH)
    dst = jnp.concatenate([edge_index[1], jnp.full((pad,), N, jnp.int32)]).reshape(NB, CH)
    eflat = jnp.concatenate(
        [jnp.stack([src, dst], 1), jnp.stack([src + NP, dst], 1)], axis=0)
    zeros = jnp.zeros((NP, DIN), jnp.float32)

    p = _sc_agg1(x, eflat, zeros)
    h1 = _tc_mlp1(x, p, c1_w1, c1_b1.reshape(1, H), c1_w2, c1_b2.reshape(1, H))
    a2 = _sc_agg2(h1.reshape(2 * NP, DIN), eflat, zeros)
    batch_r = batch.reshape(N // BN, 1, BN)
    return _tc_mlp2(
        h1, a2, batch_r,
        c2_w1, c2_b1.reshape(1, H), c2_w2, c2_b2.reshape(1, H),
        lin_w, lin_b.reshape(1, H // 2),
        lin2_w, lin2_b.reshape(1, H // 2),
        lin3_w, lin3_b.reshape(1, NCLS),
    )


# speedup vs baseline: 1.5578x; 1.0242x over previous
import functools

import jax
import jax.numpy as jnp
from jax import lax
from jax.experimental import pallas as pl
from jax.experimental.pallas import tpu as pltpu
from jax.experimental.pallas import tpu_sc as plsc

N = 10000
NP = 10240
E = 320000
G = 64
DIN = 128
H = 256
NCLS = 10

CH = 80
EP = 320000
NB = EP // CH
CORES = 2
SUBC = 16
BN = 1000

_MESH = plsc.VectorSubcoreMesh(
    core_axis_name="c", subcore_axis_name="s", num_cores=CORES, num_subcores=SUBC
)


def _make_sc_agg(table_len, chunks_per_tile, core_offset):
    npt = chunks_per_tile

    @functools.partial(
        pl.kernel,
        out_type=jax.ShapeDtypeStruct((CORES, NP, DIN), jnp.float32),
        mesh=_MESH,
        scratch_types=[
            pltpu.VMEM_SHARED((NP, DIN), jnp.float32),
            pltpu.VMEM((CH,), jnp.int32),
            pltpu.VMEM((CH,), jnp.int32),
            pltpu.VMEM((CH, DIN), jnp.float32),
            pltpu.SemaphoreType.DMA,
        ],
    )
    def k(tab_hbm, e_hbm, z_hbm, out_hbm, acc, sbuf, dbuf, rows, sem):
        c = lax.axis_index("c")
        s = lax.axis_index("s")
        rpt = NP // SUBC
        pltpu.sync_copy(z_hbm.at[pl.ds(s * rpt, rpt)], acc.at[pl.ds(s * rpt, rpt)])
        if core_offset:
            base = s * npt
            soff = c * NB
        else:
            base = (c * SUBC + s) * npt
            soff = 0
        plsc.subcore_barrier()

        def step(j, carry):
            row = base + j
            pltpu.sync_copy(e_hbm.at[soff + row], sbuf)
            pltpu.sync_copy(e_hbm.at[2 * NB + row], dbuf)
            pltpu.async_copy(tab_hbm.at[sbuf], rows, sem).wait()
            pltpu.sync_copy(rows, acc.at[dbuf], add=True)
            return carry

        lax.fori_loop(0, npt, step, 0)
        plsc.subcore_barrier()
        pltpu.sync_copy(acc.at[pl.ds(s * rpt, rpt)], out_hbm.at[c].at[pl.ds(s * rpt, rpt)])

    return k


_sc_agg1 = _make_sc_agg(NP, NB // (CORES * SUBC), core_offset=False)
_sc_agg2 = _make_sc_agg(2 * NP, NB // SUBC, core_offset=True)


def _mlp1_body(x_ref, p_ref, w1_ref, b1_ref, w2_ref, b2_ref, o_ref):
    h = x_ref[...] + p_ref[0] + p_ref[1]
    t = jnp.dot(h, w1_ref[...], preferred_element_type=jnp.float32) + b1_ref[...]
    t = jnp.maximum(t, 0.0)
    u = jnp.dot(t, w2_ref[...], preferred_element_type=jnp.float32) + b2_ref[...]
    u = jnp.where(u >= 0, u, 0.01 * u)
    o_ref[0] = u[:, :DIN]
    o_ref[1] = u[:, DIN:]


def _tc_mlp1(x, p, w1, b1, w2, b2):
    return pl.pallas_call(
        _mlp1_body,
        grid=(N // BN,),
        in_specs=[
            pl.BlockSpec((BN, DIN), lambda i: (i, 0)),
            pl.BlockSpec((CORES, BN, DIN), lambda i: (0, i, 0)),
            pl.BlockSpec((DIN, H), lambda i: (0, 0)),
            pl.BlockSpec((1, H), lambda i: (0, 0)),
            pl.BlockSpec((H, H), lambda i: (0, 0)),
            pl.BlockSpec((1, H), lambda i: (0, 0)),
        ],
        out_specs=pl.BlockSpec((CORES, BN, DIN), lambda i: (0, i, 0)),
        out_shape=jax.ShapeDtypeStruct((CORES, NP, DIN), jnp.float32),
    )(x, p, w1, b1, w2, b2)


def _mlp2_body(h_ref, a_ref, b_ref, w1_ref, b1_ref, w2_ref, b2_ref,
               lw_ref, lb_ref, l2w_ref, l2b_ref, l3w_ref, l3b_ref,
               o_ref, sums, cnts):
    i = pl.program_id(0)
    z = jnp.concatenate([h_ref[0] + a_ref[0], h_ref[1] + a_ref[1]], axis=1)
    t = jnp.dot(z, w1_ref[...], preferred_element_type=jnp.float32) + b1_ref[...]
    t = jnp.maximum(t, 0.0)
    u = jnp.dot(t, w2_ref[...], preferred_element_type=jnp.float32) + b2_ref[...]
    u = jnp.where(u >= 0, u, 0.01 * u)
    bvec = b_ref[0, 0]
    oh = (bvec[:, None] == lax.broadcasted_iota(jnp.int32, (BN, G), 1)).astype(jnp.float32)
    ps = lax.dot_general(oh, u, (((0,), (0,)), ((), ())),
                         preferred_element_type=jnp.float32)
    pc = lax.dot_general(oh, jnp.ones((BN, 1), jnp.float32), (((0,), (0,)), ((), ())),
                         preferred_element_type=jnp.float32)

    @pl.when(i == 0)
    def _():
        sums[...] = ps
        cnts[...] = pc

    @pl.when(i != 0)
    def _():
        sums[...] += ps
        cnts[...] += pc

    @pl.when(i == pl.num_programs(0) - 1)
    def _():
        cnt = jnp.maximum(cnts[...], 1.0)
        mean = sums[...] / cnt
        q = jnp.dot(mean, lw_ref[...], preferred_element_type=jnp.float32) + lb_ref[...]
        q = jnp.where(q >= 0, q, 0.01 * q)
        q = jnp.dot(q, l2w_ref[...], preferred_element_type=jnp.float32) + l2b_ref[...]
        q = jnp.where(q >= 0, q, 0.01 * q)
        o_ref[...] = jnp.dot(q, l3w_ref[...], preferred_element_type=jnp.float32) + l3b_ref[...]


def _tc_mlp2(h, a, batch_r, w1, b1, w2, b2, lw, lb, l2w, l2b, l3w, l3b):
    full = lambda shp: pl.BlockSpec(shp, lambda i: tuple(0 for _ in shp))
    return pl.pallas_call(
        _mlp2_body,
        grid=(N // BN,),
        in_specs=[
            pl.BlockSpec((CORES, BN, DIN), lambda i: (0, i, 0)),
            pl.BlockSpec((CORES, BN, DIN), lambda i: (0, i, 0)),
            pl.BlockSpec((1, 1, BN), lambda i: (i, 0, 0)),
            full((H, H)), full((1, H)), full((H, H)), full((1, H)),
            full((H, H // 2)), full((1, H // 2)),
            full((H // 2, H // 2)), full((1, H // 2)),
            full((H // 2, NCLS)), full((1, NCLS)),
        ],
        out_specs=pl.BlockSpec((G, NCLS), lambda i: (0, 0)),
        out_shape=jax.ShapeDtypeStruct((G, NCLS), jnp.float32),
        scratch_shapes=[
            pltpu.VMEM((G, H), jnp.float32),
            pltpu.VMEM((G, 1), jnp.float32),
        ],
    )(h, a, batch_r, w1, b1, w2, b2, lw, lb, l2w, l2b, l3w, l3b)


def kernel(x, edge_index, batch, c1_w1, c1_b1, c1_w2, c1_b2,
           c2_w1, c2_b1, c2_w2, c2_b2, lin_w, lin_b, lin2_w, lin2_b,
           lin3_w, lin3_b):
    src = edge_index[0].reshape(NB, CH)
    dst = edge_index[1].reshape(NB, CH)
    eflat = jnp.concatenate([src, src + NP, dst], axis=0)
    zeros = jnp.zeros((NP, DIN), jnp.float32)

    p = _sc_agg1(x, eflat, zeros)
    h1 = _tc_mlp1(x, p, c1_w1, c1_b1.reshape(1, H), c1_w2, c1_b2.reshape(1, H))
    a2 = _sc_agg2(h1.reshape(2 * NP, DIN), eflat, zeros)
    batch_r = batch.reshape(N // BN, 1, BN)
    return _tc_mlp2(
        h1, a2, batch_r,
        c2_w1, c2_b1.reshape(1, H), c2_w2, c2_b2.reshape(1, H),
        lin_w, lin_b.reshape(1, H // 2),
        lin2_w, lin2_b.reshape(1, H // 2),
        lin3_w, lin3_b.reshape(1, NCLS),
    )
